# baseline (speedup 1.0000x reference)
import functools

import jax
import jax.numpy as jnp
from jax import lax
from jax.experimental import pallas as pl
from jax.experimental.pallas import tpu as pltpu
from jax.experimental.pallas import tpu_sc as plsc

N_FACTORS = 16
BATCH = 16384
NUM_WORKERS = 32
B_PER_W = BATCH // NUM_WORKERS
CCHUNK = 16
N_CCHUNKS = B_PER_W // CCHUNK


def _body(u_hbm, i_hbm, ut_hbm, it_hbm, out_hbm,
          idx_u, idx_i, vals_u, vals_i, out_v, sem):
    wid = lax.axis_index("s") * 2 + lax.axis_index("c")
    base = pl.multiple_of(wid * B_PER_W, B_PER_W)

    pltpu.sync_copy(u_hbm.at[pl.ds(base, B_PER_W)], idx_u)
    pltpu.sync_copy(i_hbm.at[pl.ds(base, B_PER_W)], idx_i)

    def issue(cidx, _):
        csl = pl.ds(pl.multiple_of(cidx * CCHUNK, CCHUNK), CCHUNK)
        uvec = idx_u[csl]
        ivec = idx_i[csl]
        for d in range(N_FACTORS):
            pltpu.async_copy(ut_hbm.at[d].at[uvec], vals_u.at[d, csl], sem)
            pltpu.async_copy(it_hbm.at[d].at[ivec], vals_i.at[d, csl], sem)
        return _

    lax.fori_loop(0, N_CCHUNKS, issue, None)

    def drain_compute(cidx, _):
        csl = pl.ds(pl.multiple_of(cidx * CCHUNK, CCHUNK), CCHUNK)
        uvec = idx_u[csl]
        ivec = idx_i[csl]
        for d in range(N_FACTORS):
            pltpu.make_async_copy(
                ut_hbm.at[d].at[uvec], vals_u.at[d, csl], sem).wait()
            pltpu.make_async_copy(
                it_hbm.at[d].at[ivec], vals_i.at[d, csl], sem).wait()
        acc = jnp.zeros((CCHUNK,), jnp.float32)
        for d in range(N_FACTORS):
            acc = acc + vals_u[d, csl] * vals_i[d, csl]
        out_v[csl] = acc
        return _

    lax.fori_loop(0, N_CCHUNKS, drain_compute, None)

    pltpu.sync_copy(out_v, out_hbm.at[pl.ds(base, B_PER_W)])


@jax.jit
def kernel(u, i, user_emb, item_emb):
    mesh = plsc.VectorSubcoreMesh(core_axis_name="c", subcore_axis_name="s")
    run = pl.kernel(
        _body,
        mesh=mesh,
        out_type=jax.ShapeDtypeStruct((BATCH,), jnp.float32),
        scratch_types=[
            pltpu.VMEM((B_PER_W,), jnp.int32),
            pltpu.VMEM((B_PER_W,), jnp.int32),
            pltpu.VMEM((N_FACTORS, B_PER_W), jnp.float32),
            pltpu.VMEM((N_FACTORS, B_PER_W), jnp.float32),
            pltpu.VMEM((B_PER_W,), jnp.float32),
            pltpu.SemaphoreType.DMA,
        ],
        compiler_params=pltpu.CompilerParams(
            needs_layout_passes=False, use_tc_tiling_on_sc=False),
    )
    return run(u, i, user_emb.T, item_emb.T)
